# Initial kernel scaffold; baseline (speedup 1.0000x reference)
import jax
import jax.numpy as jnp
from jax.experimental import pallas as pl


def kernel(hidden_states, router_w, gate_w, up_w, down_w):
    raise NotImplementedError("write your pallas kernel here")


# speedup vs baseline: 1.4874x; 1.4874x over previous
import functools

import jax
import jax.numpy as jnp
from jax import lax
from jax.experimental import pallas as pl
from jax.experimental.pallas import tpu as pltpu


def _router_body(x_ref, rw_ref, logits_ref, w_ref, *, num_experts):
    xf = x_ref[...]
    logits = lax.dot_general(
        xf, rw_ref[...], (((1,), (1,)), ((), ())),
        preferred_element_type=jnp.float32)
    logits_ref[...] = logits
    m_tok = logits.shape[0]
    lane = lax.broadcasted_iota(jnp.int32, (m_tok, num_experts), 1)
    neg = jnp.float32(-jnp.inf)
    mx = jnp.max(logits, axis=1, keepdims=True)
    ex = jnp.exp(logits - mx)
    p = ex / jnp.sum(ex, axis=1, keepdims=True)
    m1 = jnp.max(p, axis=1, keepdims=True)
    a1 = jnp.min(jnp.where(p == m1, lane, num_experts),
                 axis=1, keepdims=True)
    p2 = jnp.where(lane == a1, neg, p)
    m2 = jnp.max(p2, axis=1, keepdims=True)
    a2 = jnp.min(jnp.where(p2 == m2, lane, num_experts),
                 axis=1, keepdims=True)
    inv = 1.0 / (m1 + m2)
    w_ref[...] = jnp.where(
        lane == a1, m1 * inv, jnp.where(lane == a2, m2 * inv, 0.0))


def _moe_body(x_ref, w_ref, gate_ref, up_ref, down_ref, out_ref):
    e = pl.program_id(1)
    fb = pl.program_id(2)
    first = jnp.logical_and(e == 0, fb == 0)

    xb = x_ref[...]
    gate = gate_ref[0].astype(jnp.bfloat16)
    up = up_ref[0].astype(jnp.bfloat16)
    down = down_ref[0].astype(jnp.bfloat16)
    dn = (((1,), (1,)), ((), ()))
    g = lax.dot_general(xb, gate, dn, preferred_element_type=jnp.float32)
    u = lax.dot_general(xb, up, dn, preferred_element_type=jnp.float32)
    h = (g * (1.0 / (1.0 + jnp.exp(-g))) * u).astype(jnp.bfloat16)
    y = lax.dot_general(h, down, dn, preferred_element_type=jnp.float32)

    m_tok, e_num = w_ref.shape
    lane = lax.broadcasted_iota(jnp.int32, (m_tok, e_num), 1)
    w_e = jnp.sum(jnp.where(lane == e, w_ref[...], 0.0),
                  axis=1, keepdims=True)

    @pl.when(first)
    def _init():
        out_ref[...] = y * w_e

    @pl.when(jnp.logical_not(first))
    def _acc():
        out_ref[...] += y * w_e


@functools.partial(jax.jit,
                   static_argnames=("block_m", "block_f", "interpret"))
def _moe(x, router_w, gate_w, up_w, down_w, *,
         block_m=1024, block_f=256, interpret=False):
    t, d = x.shape
    e_num, f, _ = gate_w.shape

    logits, w_full = pl.pallas_call(
        functools.partial(_router_body, num_experts=e_num),
        out_shape=[
            jax.ShapeDtypeStruct((t, e_num), jnp.float32),
            jax.ShapeDtypeStruct((t, e_num), jnp.float32),
        ],
        interpret=interpret,
    )(x, router_w)

    x16 = x.astype(jnp.bfloat16)
    grid = (t // block_m, e_num, f // block_f)
    out = pl.pallas_call(
        _moe_body,
        grid=grid,
        in_specs=[
            pl.BlockSpec((block_m, d), lambda i, e, fb: (i, 0)),
            pl.BlockSpec((block_m, e_num), lambda i, e, fb: (i, 0)),
            pl.BlockSpec((1, block_f, d), lambda i, e, fb: (e, fb, 0)),
            pl.BlockSpec((1, block_f, d), lambda i, e, fb: (e, fb, 0)),
            pl.BlockSpec((1, d, block_f), lambda i, e, fb: (e, 0, fb)),
        ],
        out_specs=pl.BlockSpec((block_m, d), lambda i, e, fb: (i, 0)),
        out_shape=jax.ShapeDtypeStruct((t, d), jnp.float32),
        compiler_params=pltpu.CompilerParams(
            dimension_semantics=("arbitrary", "arbitrary", "arbitrary")),
        interpret=interpret,
    )(x16, w_full, gate_w, up_w, down_w)
    return out, logits


def kernel(hidden_states, router_w, gate_w, up_w, down_w):
    b, s, d = hidden_states.shape
    x = hidden_states.reshape(-1, d)
    out, logits = _moe(x, router_w, gate_w, up_w, down_w)
    return out.reshape(b, s, d), logits
